# baseline (speedup 1.0000x reference)
import jax
import jax.numpy as jnp
from jax import lax
from jax.experimental import pallas as pl
from jax.experimental.pallas import tpu as pltpu, tpu_sc as plsc

D = 64
NC, NS = 2, 16
NW = NC * NS
BS = 128
SCALE = 8.0


def _body(xt_hbm, table2_hbm, out_hbm, idx_v, sidx_v, gbuf, obuf, gsem):
    wid = lax.axis_index("s") * NC + lax.axis_index("c")
    nt, ntok = xt_hbm.shape
    nj = ntok // BS
    nch = nt * nj // NW

    def block(k, carry):
        c = wid * nch + k
        t = c // nj
        j = c - t * nj
        pltpu.sync_copy(xt_hbm.at[t, pl.ds(j * BS, BS)], idx_v)
        for g in range(BS // 16):
            sl = pl.ds(g * 16, 16)
            sidx_v[sl] = jax.lax.shift_right_logical(idx_v[sl], 1)
        pltpu.async_copy(table2_hbm.at[sidx_v], gbuf, gsem).wait()

        rows = []
        cols0 = []
        for g in range(BS // 16):
            sl = pl.ds(g * 16, 16)
            rows.append(lax.iota(jnp.int32, 16) + g * 16)
            cols0.append((idx_v[sl] & 1) * D)

        @plsc.parallel_loop(0, D, step=1, unroll=2)
        def transform(d):
            for g in range(BS // 16):
                vals = plsc.load_gather(gbuf, [rows[g], cols0[g] + d])
                obuf[d, pl.ds(g * 16, 16)] = vals * SCALE

        pltpu.sync_copy(obuf, out_hbm.at[t, :, pl.ds(j * BS, BS)])
        return carry

    lax.fori_loop(0, nch, block, 0)


@jax.jit
def _embed(xt, table2):
    nt, ntok = xt.shape
    mesh = plsc.VectorSubcoreMesh(core_axis_name="c", subcore_axis_name="s")
    return pl.kernel(
        _body,
        out_type=jax.ShapeDtypeStruct((nt, D, ntok), jnp.float32),
        mesh=mesh,
        scratch_types=(
            pltpu.VMEM((BS,), jnp.int32),
            pltpu.VMEM((BS,), jnp.int32),
            pltpu.VMEM((BS, 2 * D), jnp.float32),
            pltpu.VMEM((D, BS), jnp.float32),
            pltpu.SemaphoreType.DMA,
        ),
        compiler_params=pltpu.CompilerParams(
            use_tc_tiling_on_sc=True, needs_layout_passes=False),
    )(xt, table2)


def kernel(x, table):
    out_phys = _embed(x.T, table.reshape(-1, 2 * D))
    return jnp.transpose(out_phys, (2, 0, 1))


# speedup vs baseline: 1.3499x; 1.3499x over previous
import jax
import jax.numpy as jnp
from jax import lax
from jax.experimental import pallas as pl
from jax.experimental.pallas import tpu as pltpu, tpu_sc as plsc

D = 64
NC, NS = 2, 16
NW = NC * NS
BS = 128
NB = 4
SCALE = 8.0


def _body(xt_hbm, tab_hbm, out_hbm, *scratch):
    idxs = scratch[:NB]
    sidxs = scratch[NB:2 * NB]
    cols = scratch[2 * NB:3 * NB]
    gbufs = scratch[3 * NB:4 * NB]
    obufs = scratch[4 * NB:5 * NB]
    isems = scratch[5 * NB:6 * NB]
    gsems = scratch[6 * NB:7 * NB]
    osems = scratch[7 * NB:8 * NB]
    wid = lax.axis_index("s") * NC + lax.axis_index("c")
    nt, ntok = xt_hbm.shape
    nj = ntok // BS
    nch = nt * nj // NW
    c0 = wid * nch

    def tj(c):
        t = (c0 + c) // nj
        return t, (c0 + c) - t * nj

    def idx_dma(c, s):
        t, j = tj(c)
        return pltpu.make_async_copy(
            xt_hbm.at[t, pl.ds(j * BS, BS)], idxs[s], isems[s])

    def gat_dma(c, s):
        del c
        return pltpu.make_async_copy(tab_hbm.at[sidxs[s]], gbufs[s], gsems[s])

    def shift(s):
        for g in range(BS // 16):
            sl = pl.ds(g * 16, 16)
            iv = idxs[s][sl]
            sidxs[s][sl] = lax.shift_right_logical(iv, 1)
            cols[s][sl] = (iv & 1) * D

    def out_dma(c, s):
        t, j = tj(c)
        return pltpu.make_async_copy(
            obufs[s], out_hbm.at[t, :, pl.ds(j * BS, BS)], osems[s])

    rows = [lax.iota(jnp.int32, 16) + g * 16 for g in range(BS // 16)]

    for c in range(NB):
        idx_dma(c, c % NB).start()
    for c in range(NB - 1):
        idx_dma(c, c % NB).wait()
        shift(c % NB)
        gat_dma(c, c % NB).start()
    idx_dma(NB, 0).start()

    def step(o, carry):
        for b in range(NB):
            c = o * NB + b

            @pl.when(c >= NB)
            def _():
                out_dma(c - NB, b).wait()

            f = c + NB - 1
            fs = (b + NB - 1) % NB

            @pl.when(f < nch)
            def _():
                idx_dma(f, fs).wait()
                shift(fs)
                gat_dma(f, fs).start()

            f2 = c + NB + 1
            fs2 = (b + 1) % NB

            @pl.when(f2 < nch)
            def _():
                idx_dma(f2, fs2).start()

            gat_dma(c, b).wait()
            gbuf, obuf = gbufs[b], obufs[b]
            cvecs = [cols[b][pl.ds(g * 16, 16)] for g in range(BS // 16)]

            @plsc.parallel_loop(0, D, step=1, unroll=2)
            def transform(d):
                for g in range(BS // 16):
                    vals = plsc.load_gather(gbuf, [rows[g], cvecs[g] + d])
                    obuf[d, pl.ds(g * 16, 16)] = vals * SCALE

            out_dma(c, b).start()
        return carry

    lax.fori_loop(0, nch // NB, step, 0)
    for k in range(NB):
        c = nch - NB + k
        out_dma(c, c % NB).wait()


@jax.jit
def _embed(xt, tab):
    nt, ntok = xt.shape
    mesh = plsc.VectorSubcoreMesh(core_axis_name="c", subcore_axis_name="s")
    return pl.kernel(
        _body,
        out_type=jax.ShapeDtypeStruct((nt, D, ntok), jnp.float32),
        mesh=mesh,
        scratch_types=(
            [pltpu.VMEM((BS,), jnp.int32) for _ in range(3 * NB)]
            + [pltpu.VMEM((BS, 2 * D), jnp.float32) for _ in range(NB)]
            + [pltpu.VMEM((D, BS), jnp.float32) for _ in range(NB)]
            + [pltpu.SemaphoreType.DMA for _ in range(3 * NB)]
        ),
        compiler_params=pltpu.CompilerParams(
            use_tc_tiling_on_sc=True, needs_layout_passes=False),
    )(xt, tab)


def kernel(x, table):
    out_phys = _embed(x.T, table.reshape(-1, 2 * D))
    return jnp.transpose(out_phys, (2, 0, 1))
